# baseline (speedup 1.0000x reference)
import jax
import jax.numpy as jnp
from jax import lax
from jax.experimental import pallas as pl
from jax.experimental.pallas import tpu as pltpu
from jax.experimental.pallas import tpu_sc as plsc

VOCAB = 101
D_MODEL = 1024
B_TOTAL = 4 * 4096

_INFO = plsc.get_sparse_core_info()
_NC, _NS, _NL = _INFO.num_cores, _INFO.num_subcores, _INFO.num_lanes
_NW = _NC * _NS
_BPW = B_TOTAL // _NW
_WIN = 64


def _body(table_hbm, ids_hbm, out_hbm, table_v, idx_v, wsem):
    wid = lax.axis_index("s") * _NC + lax.axis_index("c")
    base = wid * _BPW
    pltpu.sync_copy(table_hbm, table_v)
    pltpu.sync_copy(ids_hbm.at[pl.ds(base, _BPW)], idx_v)

    def row_copy(pos, rid):
        return pltpu.make_async_copy(
            table_v.at[pl.ds(rid, 1)], out_hbm.at[pl.ds(pos, 1)], wsem)

    def fire(i, carry):
        ridv = plsc.load_gather(idx_v, [jnp.broadcast_to(i, (_NL,))])
        rid = lax.reduce_max(ridv, (0,))
        row_copy(base + i, rid).start()

        @pl.when(i >= _WIN)
        def _():
            row_copy(base, 0).wait()

        return carry

    lax.fori_loop(0, _BPW, fire, 0)

    def drain(i, carry):
        row_copy(base, 0).wait()
        return carry

    lax.fori_loop(0, _WIN, drain, 0)


@jax.jit
def _run(ids_flat, embeddings):
    mesh = plsc.VectorSubcoreMesh(core_axis_name="c", subcore_axis_name="s")
    k = pl.kernel(
        _body,
        out_type=jax.ShapeDtypeStruct((B_TOTAL, D_MODEL), jnp.float32),
        mesh=mesh,
        scratch_types=[
            pltpu.VMEM((VOCAB, D_MODEL), jnp.float32),
            pltpu.VMEM((_BPW,), jnp.int32),
            pltpu.SemaphoreType.DMA,
        ],
        compiler_params=pltpu.CompilerParams(needs_layout_passes=False),
    )
    return k(embeddings, ids_flat)


def kernel(lang_ids, embeddings):
    ids_flat = lang_ids.reshape(-1)
    if ids_flat.dtype != jnp.int32:
        ids_flat = ids_flat.astype(jnp.int32)
    out = _run(ids_flat, embeddings)
    return out.reshape(lang_ids.shape + (D_MODEL,))
